# baseline (speedup 1.0000x reference)
import jax
import jax.numpy as jnp
from jax.experimental import pallas as pl
from jax.experimental.pallas import tpu as pltpu

M = 1024
K = 48000
KT = 9600
MT = 512
NK = K // KT
NM = M // MT


def _fused_body(x_ref, w1_ref, b1_ref, w2_ref, b2_ref, w3_ref, b3_ref,
                w4_ref, b4_ref, w5_ref, b5_ref, out_ref, acc_ref):
    k = pl.program_id(0)
    m = pl.program_id(1)

    part = jax.lax.dot_general(
        x_ref[...], w1_ref[...],
        dimension_numbers=(((1,), (1,)), ((), ())),
        preferred_element_type=jnp.float32)
    rows = pl.ds(m * MT, MT)

    @pl.when(k == 0)
    def _init():
        acc_ref[rows, :] = part

    @pl.when(k > 0)
    def _accum():
        acc_ref[rows, :] += part

    @pl.when(k == NK - 1)
    def _epilogue():
        h = acc_ref[rows, :] + b1_ref[...]
        h = h * jax.nn.sigmoid(h)
        h = jax.lax.dot_general(
            h, w2_ref[...], dimension_numbers=(((1,), (1,)), ((), ())),
            preferred_element_type=jnp.float32) + b2_ref[...]
        h = h * jax.nn.sigmoid(h)
        h = jax.lax.dot_general(
            h, w3_ref[...], dimension_numbers=(((1,), (1,)), ((), ())),
            preferred_element_type=jnp.float32) + b3_ref[...]
        h = h * jax.nn.sigmoid(h)
        h = jax.lax.dot_general(
            h, w4_ref[...], dimension_numbers=(((1,), (1,)), ((), ())),
            preferred_element_type=jnp.float32) + b4_ref[...]
        h = h * jax.nn.sigmoid(h)
        h = jax.lax.dot_general(
            h, w5_ref[...], dimension_numbers=(((1,), (1,)), ((), ())),
            preferred_element_type=jnp.float32) + b5_ref[...]
        out_ref[...] = h


def kernel(x, W1, b1, W2, b2, W3, b3, W4, b4, W5, b5):
    b1r = b1.reshape(1, -1)
    b2r = b2.reshape(1, -1)
    b3r = b3.reshape(1, -1)
    b4r = b4.reshape(1, -1)
    b5r = b5.reshape(1, -1)

    def _const(shape):
        return pl.BlockSpec(shape, lambda k, m: (0, 0))

    return pl.pallas_call(
        _fused_body,
        grid=(NK, NM),
        in_specs=[
            pl.BlockSpec((MT, KT), lambda k, m: (m, k)),
            pl.BlockSpec((W1.shape[0], KT), lambda k, m: (0, k)),
            _const(b1r.shape),
            _const(W2.shape),
            _const(b2r.shape),
            _const(W3.shape),
            _const(b3r.shape),
            _const(W4.shape),
            _const(b4r.shape),
            _const(W5.shape),
            _const(b5r.shape),
        ],
        out_specs=pl.BlockSpec((MT, W5.shape[0]), lambda k, m: (m, 0)),
        out_shape=jax.ShapeDtypeStruct((M, W5.shape[0]), jnp.float32),
        scratch_shapes=[pltpu.VMEM((M, W1.shape[0]), jnp.float32)],
        compiler_params=pltpu.CompilerParams(
            dimension_semantics=("arbitrary", "arbitrary"),
        ),
    )(x, W1, b1r, W2, b2r, W3, b3r, W4, b4r, W5, b5r)


# speedup vs baseline: 1.0439x; 1.0439x over previous
import jax
import jax.numpy as jnp
from jax.experimental import pallas as pl
from jax.experimental.pallas import tpu as pltpu

M = 1024
K = 48000
KT = 3200
NSTEPS = K // KT


def _fused_body(x_ref, w1_ref, b1_ref, w2_ref, b2_ref, w3_ref, b3_ref,
                w4_ref, b4_ref, w5_ref, b5_ref, out_ref, acc_ref):
    k = pl.program_id(0)

    part = jax.lax.dot_general(
        x_ref[...], w1_ref[:, pl.ds(k * KT, KT)],
        dimension_numbers=(((1,), (1,)), ((), ())),
        preferred_element_type=jnp.float32)

    @pl.when(k == 0)
    def _init():
        acc_ref[...] = part

    @pl.when(k > 0)
    def _accum():
        acc_ref[...] += part

    @pl.when(k == NSTEPS - 1)
    def _epilogue():
        h = acc_ref[...] + b1_ref[...]
        h = h * jax.nn.sigmoid(h)
        h = jax.lax.dot_general(
            h, w2_ref[...], dimension_numbers=(((1,), (1,)), ((), ())),
            preferred_element_type=jnp.float32) + b2_ref[...]
        h = h * jax.nn.sigmoid(h)
        h = jax.lax.dot_general(
            h, w3_ref[...], dimension_numbers=(((1,), (1,)), ((), ())),
            preferred_element_type=jnp.float32) + b3_ref[...]
        h = h * jax.nn.sigmoid(h)
        h = jax.lax.dot_general(
            h, w4_ref[...], dimension_numbers=(((1,), (1,)), ((), ())),
            preferred_element_type=jnp.float32) + b4_ref[...]
        h = h * jax.nn.sigmoid(h)
        h = jax.lax.dot_general(
            h, w5_ref[...], dimension_numbers=(((1,), (1,)), ((), ())),
            preferred_element_type=jnp.float32) + b5_ref[...]
        out_ref[...] = h


def kernel(x, W1, b1, W2, b2, W3, b3, W4, b4, W5, b5):
    b1r = b1.reshape(1, -1)
    b2r = b2.reshape(1, -1)
    b3r = b3.reshape(1, -1)
    b4r = b4.reshape(1, -1)
    b5r = b5.reshape(1, -1)

    def _const(shape):
        return pl.BlockSpec(shape, lambda k: (0, 0))

    return pl.pallas_call(
        _fused_body,
        grid=(NSTEPS,),
        in_specs=[
            pl.BlockSpec((M, KT), lambda k: (0, k)),
            _const(W1.shape),
            _const(b1r.shape),
            _const(W2.shape),
            _const(b2r.shape),
            _const(W3.shape),
            _const(b3r.shape),
            _const(W4.shape),
            _const(b4r.shape),
            _const(W5.shape),
            _const(b5r.shape),
        ],
        out_specs=pl.BlockSpec((M, W5.shape[0]), lambda k: (0, 0)),
        out_shape=jax.ShapeDtypeStruct((M, W5.shape[0]), jnp.float32),
        scratch_shapes=[pltpu.VMEM((M, W1.shape[0]), jnp.float32)],
        compiler_params=pltpu.CompilerParams(
            dimension_semantics=("arbitrary",),
        ),
    )(x, W1, b1r, W2, b2r, W3, b3r, W4, b4r, W5, b5r)
